# baseline (speedup 1.0000x reference)
import functools

import numpy as np
import jax
import jax.numpy as jnp
from jax import lax
from jax.experimental import pallas as pl
from jax.experimental.pallas import tpu as pltpu
from jax.experimental.pallas import tpu_sc as plsc

N = 10000
NPAD = 10240
D_IN = 128
HID = 32
NCLS = 40
F2 = 48
EPS = 1e-5
E = 320000
CHUNK = 256
NWORK = 32
EPW = E // NWORK
CPW = 39
TAIL = EPW - CPW * CHUNK
NSUB = 16
RPS = NPAD // NSUB
DEGW = 16
NBUF = 3
ZROWS = 128
BLK = 2000

def _fill(ref, value, feat):
    @pl.loop(0, ref.shape[0])
    def _(i):
        for k in range(feat // 16):
            ref[i, pl.ds(k * 16, 16)] = jnp.full((16,), value, jnp.float32)


def _zero_acc(zbuf, acc_sh, s):
    for q in range(RPS // ZROWS):
        pltpu.sync_copy(zbuf, acc_sh.at[pl.ds(s * RPS + q * ZROWS, ZROWS)])


def _mm_scale_body(x_ref, w_ref, degp_ref, g1_ref, dinv_ref):
    h1 = jnp.dot(x_ref[...], w_ref[...], preferred_element_type=jnp.float32)
    deg = degp_ref[0, :, :1] + degp_ref[1, :, :1] + 1.0
    dinv = lax.rsqrt(deg)
    dinv_ref[...] = dinv
    g1_ref[...] = h1 * dinv


def _mm_scale(x, w, degp):
    nb = N // BLK
    return pl.pallas_call(
        _mm_scale_body,
        grid=(nb,),
        in_specs=[pl.BlockSpec((BLK, D_IN), lambda i: (i, 0)),
                  pl.BlockSpec((D_IN, HID), lambda i: (0, 0)),
                  pl.BlockSpec((2, BLK, DEGW), lambda i: (0, i, 0))],
        out_specs=(pl.BlockSpec((BLK, HID), lambda i: (i, 0)),
                   pl.BlockSpec((BLK, 1), lambda i: (i, 0))),
        out_shape=(jax.ShapeDtypeStruct((N, HID), jnp.float32),
                   jax.ShapeDtypeStruct((N, 1), jnp.float32)),
    )(x, w, degp)


def _mid_body(accp_ref, g1_ref, dinv_ref, b1_ref, gam_ref, bet_ref, w2_ref,
              g2_ref):
    s = accp_ref[0] + accp_ref[1] + g1_ref[...]
    h = s * dinv_ref[...] + b1_ref[...]
    mu = jnp.mean(h, axis=0, keepdims=True)
    var = jnp.mean((h - mu) ** 2, axis=0, keepdims=True)
    hn = (h - mu) * lax.rsqrt(var + EPS) * gam_ref[...] + bet_ref[...]
    hr = jnp.maximum(hn, 0.0)
    h2 = jnp.dot(hr, w2_ref[...], preferred_element_type=jnp.float32)
    g2_ref[...] = h2 * dinv_ref[...]


def _mid(accp, g1, dinv, b1r, gammar, betar, w2p):
    return pl.pallas_call(
        _mid_body,
        out_shape=jax.ShapeDtypeStruct((N, F2), jnp.float32),
    )(accp, g1, dinv, b1r, gammar, betar, w2p)


def _final_body(accp_ref, g2_ref, dinv_ref, b2_ref, o_ref):
    s = accp_ref[0] + accp_ref[1] + g2_ref[...]
    res = s * dinv_ref[...] + b2_ref[...]
    o_ref[...] = res[:, :NCLS]


def _final(accp, g2, dinv, b2r):
    nb = N // BLK
    blk = lambda i: (i, 0)
    return pl.pallas_call(
        _final_body,
        grid=(nb,),
        in_specs=[pl.BlockSpec((2, BLK, F2), lambda i: (0, i, 0)),
                  pl.BlockSpec((BLK, F2), blk),
                  pl.BlockSpec((BLK, 1), blk),
                  pl.BlockSpec((1, F2), lambda i: (0, 0))],
        out_specs=pl.BlockSpec((BLK, NCLS), blk),
        out_shape=jax.ShapeDtypeStruct((N, NCLS), jnp.float32),
    )(accp, g2, dinv, b2r)


def _copy_out(acc_sh, out_hbm, c, s):
    @pl.when(s < NSUB - 1)
    def _():
        pltpu.sync_copy(acc_sh.at[pl.ds(s * RPS, RPS)],
                        out_hbm.at[c].at[pl.ds(s * RPS, RPS)])

    @pl.when(s == NSUB - 1)
    def _():
        pltpu.sync_copy(acc_sh.at[pl.ds((NSUB - 1) * RPS, N - (NSUB - 1) * RPS)],
                        out_hbm.at[c].at[pl.ds((NSUB - 1) * RPS,
                                               N - (NSUB - 1) * RPS)])


def _sc_degree(ei):
    mesh = plsc.VectorSubcoreMesh(core_axis_name="c", subcore_axis_name="s")

    @functools.partial(
        pl.kernel,
        out_type=jax.ShapeDtypeStruct((2, N, DEGW), jnp.float32),
        mesh=mesh,
        compiler_params=pltpu.CompilerParams(use_tc_tiling_on_sc=False),
        scratch_types=[
            pltpu.VMEM((EPW,), jnp.int32),
            pltpu.VMEM((CHUNK, DEGW), jnp.float32),
            pltpu.VMEM((ZROWS, DEGW), jnp.float32),
            pltpu.VMEM_SHARED((NPAD, DEGW), jnp.float32),
            pltpu.SemaphoreType.DMA,
        ],
    )
    def k(ei_hbm, out_hbm, idx_v, ones_v, zbuf_v, acc_sh, sem):
        c = lax.axis_index("c")
        s = lax.axis_index("s")
        w = c * NSUB + s
        pltpu.async_copy(ei_hbm.at[1].at[pl.ds(w * EPW, EPW)], idx_v, sem)
        _fill(ones_v, 1.0, DEGW)
        _fill(zbuf_v, 0.0, DEGW)
        _zero_acc(zbuf_v, acc_sh, s)
        pltpu.make_async_copy(ei_hbm.at[1].at[pl.ds(w * EPW, EPW)], idx_v,
                              sem).wait()
        plsc.subcore_barrier()

        @pl.loop(0, CPW, step=3)
        def _(j0):
            for u in range(3):
                pltpu.async_copy(
                    ones_v, acc_sh.at[idx_v.at[pl.ds((j0 + u) * CHUNK, CHUNK)]],
                    sem, add=True)
            for _u in range(3):
                pltpu.make_async_copy(
                    ones_v, acc_sh.at[idx_v.at[pl.ds(j0 * CHUNK, CHUNK)]],
                    sem).wait()

        pltpu.sync_copy(ones_v.at[pl.ds(0, TAIL)],
                        acc_sh.at[idx_v.at[pl.ds(CPW * CHUNK, TAIL)]],
                        add=True)
        plsc.subcore_barrier()
        _copy_out(acc_sh, out_hbm, c, s)

    return k(ei)


def _sc_edge(g, ei, feat):
    mesh = plsc.VectorSubcoreMesh(core_axis_name="c", subcore_axis_name="s")

    @functools.partial(
        pl.kernel,
        out_type=jax.ShapeDtypeStruct((2, N, feat), jnp.float32),
        mesh=mesh,
        compiler_params=pltpu.CompilerParams(use_tc_tiling_on_sc=False),
        scratch_types=[
            pltpu.VMEM((EPW,), jnp.int32),
            pltpu.VMEM((EPW,), jnp.int32),
            pltpu.VMEM((TAIL, feat), jnp.float32),
            pltpu.VMEM((ZROWS, feat), jnp.float32),
        ] + [pltpu.VMEM((CHUNK, feat), jnp.float32)] * NBUF + [
            pltpu.VMEM_SHARED((NPAD, feat), jnp.float32),
        ] + [pltpu.SemaphoreType.DMA] * (NBUF + 2),
    )
    def k(g_hbm, ei_hbm, out_hbm, src_v, dst_v, tail_v, zbuf_v, *rest):
        bufs = rest[:NBUF]
        acc_sh = rest[NBUF]
        sems = rest[NBUF + 1:NBUF + 1 + NBUF]
        isem = rest[NBUF + 1 + NBUF]
        tsem = rest[NBUF + 2 + NBUF]
        c = lax.axis_index("c")
        s = lax.axis_index("s")
        w = c * NSUB + s
        pltpu.async_copy(ei_hbm.at[0].at[pl.ds(w * EPW, EPW)], src_v, isem)
        pltpu.async_copy(ei_hbm.at[1].at[pl.ds(w * EPW, EPW)], dst_v, isem)
        _fill(zbuf_v, 0.0, feat)
        _zero_acc(zbuf_v, acc_sh, s)
        pltpu.make_async_copy(ei_hbm.at[0].at[pl.ds(w * EPW, EPW)], src_v,
                              isem).wait()
        pltpu.make_async_copy(ei_hbm.at[1].at[pl.ds(w * EPW, EPW)], dst_v,
                              isem).wait()
        plsc.subcore_barrier()

        def gidx(j):
            return src_v.at[pl.ds(j * CHUNK, CHUNK)]

        def sidx(j):
            return dst_v.at[pl.ds(j * CHUNK, CHUNK)]

        for u in range(NBUF):
            pltpu.async_copy(g_hbm.at[gidx(u)], bufs[u], sems[u])
        pltpu.async_copy(g_hbm.at[src_v.at[pl.ds(CPW * CHUNK, TAIL)]], tail_v,
                         tsem)

        @pl.loop(0, CPW - NBUF, step=NBUF)
        def _(j):
            for u in range(NBUF):
                pltpu.make_async_copy(g_hbm.at[gidx(j + u)], bufs[u],
                                      sems[u]).wait()
                pltpu.sync_copy(bufs[u], acc_sh.at[sidx(j + u)], add=True)
                pltpu.async_copy(g_hbm.at[gidx(j + NBUF + u)], bufs[u],
                                 sems[u])

        for u in range(NBUF):
            j = CPW - NBUF + u
            pltpu.make_async_copy(g_hbm.at[gidx(j)], bufs[u], sems[u]).wait()
            pltpu.sync_copy(bufs[u], acc_sh.at[sidx(j)], add=True)

        pltpu.make_async_copy(g_hbm.at[src_v.at[pl.ds(CPW * CHUNK, TAIL)]],
                              tail_v, tsem).wait()
        pltpu.sync_copy(tail_v, acc_sh.at[dst_v.at[pl.ds(CPW * CHUNK, TAIL)]],
                        add=True)

        plsc.subcore_barrier()
        _copy_out(acc_sh, out_hbm, c, s)

    return k(g, ei)


def kernel(x, edge_index, W1, b1, gamma, beta, W2, b2):
    w2p = jnp.pad(W2, ((0, 0), (0, F2 - NCLS)))
    b2r = jnp.pad(b2, (0, F2 - NCLS)).reshape(1, F2)
    b1r = b1.reshape(1, HID)
    gammar = gamma.reshape(1, HID)
    betar = beta.reshape(1, HID)

    degp = _sc_degree(edge_index)
    g1, dinv = _mm_scale(x, W1, degp)
    acc1 = _sc_edge(g1, edge_index, HID)
    g2 = _mid(acc1, g1, dinv, b1r, gammar, betar, w2p)
    acc2 = _sc_edge(g2, edge_index, F2)
    return _final(acc2, g2, dinv, b2r)


# speedup vs baseline: 1.0149x; 1.0149x over previous
import functools

import numpy as np
import jax
import jax.numpy as jnp
from jax import lax
from jax.experimental import pallas as pl
from jax.experimental.pallas import tpu as pltpu
from jax.experimental.pallas import tpu_sc as plsc

N = 10000
NPAD = 10240
D_IN = 128
HID = 32
NCLS = 40
F2 = 48
EPS = 1e-5
E = 320000
CHUNK = 128
NWORK = 32
EPW = E // NWORK
CPW = 78
TAIL = EPW - CPW * CHUNK
NSUB = 16
RPS = NPAD // NSUB
DEGW = 16
NBUF = 6
ZROWS = 128
BLK = 2000

def _fill(ref, value, feat):
    @pl.loop(0, ref.shape[0])
    def _(i):
        for k in range(feat // 16):
            ref[i, pl.ds(k * 16, 16)] = jnp.full((16,), value, jnp.float32)


def _zero_acc(zbuf, acc_sh, s):
    for q in range(RPS // ZROWS):
        pltpu.sync_copy(zbuf, acc_sh.at[pl.ds(s * RPS + q * ZROWS, ZROWS)])


def _mm_scale_body(x_ref, w_ref, degp_ref, g1_ref, dinv_ref):
    h1 = jnp.dot(x_ref[...], w_ref[...], preferred_element_type=jnp.float32)
    deg = degp_ref[0, :, :1] + degp_ref[1, :, :1] + 1.0
    dinv = lax.rsqrt(deg)
    dinv_ref[...] = dinv
    g1_ref[...] = h1 * dinv


def _mm_scale(x, w, degp):
    nb = N // BLK
    return pl.pallas_call(
        _mm_scale_body,
        grid=(nb,),
        in_specs=[pl.BlockSpec((BLK, D_IN), lambda i: (i, 0)),
                  pl.BlockSpec((D_IN, HID), lambda i: (0, 0)),
                  pl.BlockSpec((2, BLK, DEGW), lambda i: (0, i, 0))],
        out_specs=(pl.BlockSpec((BLK, HID), lambda i: (i, 0)),
                   pl.BlockSpec((BLK, 1), lambda i: (i, 0))),
        out_shape=(jax.ShapeDtypeStruct((N, HID), jnp.float32),
                   jax.ShapeDtypeStruct((N, 1), jnp.float32)),
    )(x, w, degp)


def _mid_body(accp_ref, g1_ref, dinv_ref, b1_ref, gam_ref, bet_ref, w2_ref,
              g2_ref):
    s = accp_ref[0] + accp_ref[1] + g1_ref[...]
    h = s * dinv_ref[...] + b1_ref[...]
    mu = jnp.mean(h, axis=0, keepdims=True)
    var = jnp.mean((h - mu) ** 2, axis=0, keepdims=True)
    hn = (h - mu) * lax.rsqrt(var + EPS) * gam_ref[...] + bet_ref[...]
    hr = jnp.maximum(hn, 0.0)
    h2 = jnp.dot(hr, w2_ref[...], preferred_element_type=jnp.float32)
    g2_ref[...] = h2 * dinv_ref[...]


def _mid(accp, g1, dinv, b1r, gammar, betar, w2p):
    return pl.pallas_call(
        _mid_body,
        out_shape=jax.ShapeDtypeStruct((N, F2), jnp.float32),
    )(accp, g1, dinv, b1r, gammar, betar, w2p)


def _final_body(accp_ref, g2_ref, dinv_ref, b2_ref, o_ref):
    s = accp_ref[0] + accp_ref[1] + g2_ref[...]
    res = s * dinv_ref[...] + b2_ref[...]
    o_ref[...] = res[:, :NCLS]


def _final(accp, g2, dinv, b2r):
    nb = N // BLK
    blk = lambda i: (i, 0)
    return pl.pallas_call(
        _final_body,
        grid=(nb,),
        in_specs=[pl.BlockSpec((2, BLK, F2), lambda i: (0, i, 0)),
                  pl.BlockSpec((BLK, F2), blk),
                  pl.BlockSpec((BLK, 1), blk),
                  pl.BlockSpec((1, F2), lambda i: (0, 0))],
        out_specs=pl.BlockSpec((BLK, NCLS), blk),
        out_shape=jax.ShapeDtypeStruct((N, NCLS), jnp.float32),
    )(accp, g2, dinv, b2r)


def _copy_out(acc_sh, out_hbm, c, s):
    @pl.when(s < NSUB - 1)
    def _():
        pltpu.sync_copy(acc_sh.at[pl.ds(s * RPS, RPS)],
                        out_hbm.at[c].at[pl.ds(s * RPS, RPS)])

    @pl.when(s == NSUB - 1)
    def _():
        pltpu.sync_copy(acc_sh.at[pl.ds((NSUB - 1) * RPS, N - (NSUB - 1) * RPS)],
                        out_hbm.at[c].at[pl.ds((NSUB - 1) * RPS,
                                               N - (NSUB - 1) * RPS)])


def _sc_degree(ei):
    mesh = plsc.VectorSubcoreMesh(core_axis_name="c", subcore_axis_name="s")

    @functools.partial(
        pl.kernel,
        out_type=jax.ShapeDtypeStruct((2, N, DEGW), jnp.float32),
        mesh=mesh,
        compiler_params=pltpu.CompilerParams(use_tc_tiling_on_sc=False),
        scratch_types=[
            pltpu.VMEM((EPW,), jnp.int32),
            pltpu.VMEM((CHUNK, DEGW), jnp.float32),
            pltpu.VMEM((ZROWS, DEGW), jnp.float32),
            pltpu.VMEM_SHARED((NPAD, DEGW), jnp.float32),
            pltpu.SemaphoreType.DMA,
        ],
    )
    def k(ei_hbm, out_hbm, idx_v, ones_v, zbuf_v, acc_sh, sem):
        c = lax.axis_index("c")
        s = lax.axis_index("s")
        w = c * NSUB + s
        pltpu.async_copy(ei_hbm.at[1].at[pl.ds(w * EPW, EPW)], idx_v, sem)
        _fill(ones_v, 1.0, DEGW)
        _fill(zbuf_v, 0.0, DEGW)
        _zero_acc(zbuf_v, acc_sh, s)
        pltpu.make_async_copy(ei_hbm.at[1].at[pl.ds(w * EPW, EPW)], idx_v,
                              sem).wait()
        plsc.subcore_barrier()

        @pl.loop(0, CPW, step=6)
        def _(j0):
            for u in range(6):
                pltpu.async_copy(
                    ones_v, acc_sh.at[idx_v.at[pl.ds((j0 + u) * CHUNK, CHUNK)]],
                    sem, add=True)
            for _u in range(6):
                pltpu.make_async_copy(
                    ones_v, acc_sh.at[idx_v.at[pl.ds(j0 * CHUNK, CHUNK)]],
                    sem).wait()

        pltpu.sync_copy(ones_v.at[pl.ds(0, TAIL)],
                        acc_sh.at[idx_v.at[pl.ds(CPW * CHUNK, TAIL)]],
                        add=True)
        plsc.subcore_barrier()
        _copy_out(acc_sh, out_hbm, c, s)

    return k(ei)


def _sc_edge(g, ei, feat):
    mesh = plsc.VectorSubcoreMesh(core_axis_name="c", subcore_axis_name="s")

    @functools.partial(
        pl.kernel,
        out_type=jax.ShapeDtypeStruct((2, N, feat), jnp.float32),
        mesh=mesh,
        compiler_params=pltpu.CompilerParams(use_tc_tiling_on_sc=False),
        scratch_types=[
            pltpu.VMEM((EPW,), jnp.int32),
            pltpu.VMEM((EPW,), jnp.int32),
            pltpu.VMEM((TAIL, feat), jnp.float32),
            pltpu.VMEM((ZROWS, feat), jnp.float32),
        ] + [pltpu.VMEM((CHUNK, feat), jnp.float32)] * NBUF + [
            pltpu.VMEM_SHARED((NPAD, feat), jnp.float32),
        ] + [pltpu.SemaphoreType.DMA] * (NBUF + 2),
    )
    def k(g_hbm, ei_hbm, out_hbm, src_v, dst_v, tail_v, zbuf_v, *rest):
        bufs = rest[:NBUF]
        acc_sh = rest[NBUF]
        sems = rest[NBUF + 1:NBUF + 1 + NBUF]
        isem = rest[NBUF + 1 + NBUF]
        tsem = rest[NBUF + 2 + NBUF]
        c = lax.axis_index("c")
        s = lax.axis_index("s")
        w = c * NSUB + s
        pltpu.async_copy(ei_hbm.at[0].at[pl.ds(w * EPW, EPW)], src_v, isem)
        pltpu.async_copy(ei_hbm.at[1].at[pl.ds(w * EPW, EPW)], dst_v, isem)
        _fill(zbuf_v, 0.0, feat)
        _zero_acc(zbuf_v, acc_sh, s)
        pltpu.make_async_copy(ei_hbm.at[0].at[pl.ds(w * EPW, EPW)], src_v,
                              isem).wait()
        pltpu.make_async_copy(ei_hbm.at[1].at[pl.ds(w * EPW, EPW)], dst_v,
                              isem).wait()
        plsc.subcore_barrier()

        def gidx(j):
            return src_v.at[pl.ds(j * CHUNK, CHUNK)]

        def sidx(j):
            return dst_v.at[pl.ds(j * CHUNK, CHUNK)]

        for u in range(NBUF):
            pltpu.async_copy(g_hbm.at[gidx(u)], bufs[u], sems[u])
        pltpu.async_copy(g_hbm.at[src_v.at[pl.ds(CPW * CHUNK, TAIL)]], tail_v,
                         tsem)

        @pl.loop(0, CPW - NBUF, step=NBUF)
        def _(j):
            for u in range(NBUF):
                pltpu.make_async_copy(g_hbm.at[gidx(j + u)], bufs[u],
                                      sems[u]).wait()
                pltpu.sync_copy(bufs[u], acc_sh.at[sidx(j + u)], add=True)
                pltpu.async_copy(g_hbm.at[gidx(j + NBUF + u)], bufs[u],
                                 sems[u])

        for u in range(NBUF):
            j = CPW - NBUF + u
            pltpu.make_async_copy(g_hbm.at[gidx(j)], bufs[u], sems[u]).wait()
            pltpu.sync_copy(bufs[u], acc_sh.at[sidx(j)], add=True)

        pltpu.make_async_copy(g_hbm.at[src_v.at[pl.ds(CPW * CHUNK, TAIL)]],
                              tail_v, tsem).wait()
        pltpu.sync_copy(tail_v, acc_sh.at[dst_v.at[pl.ds(CPW * CHUNK, TAIL)]],
                        add=True)

        plsc.subcore_barrier()
        _copy_out(acc_sh, out_hbm, c, s)

    return k(g, ei)


def kernel(x, edge_index, W1, b1, gamma, beta, W2, b2):
    w2p = jnp.pad(W2, ((0, 0), (0, F2 - NCLS)))
    b2r = jnp.pad(b2, (0, F2 - NCLS)).reshape(1, F2)
    b1r = b1.reshape(1, HID)
    gammar = gamma.reshape(1, HID)
    betar = beta.reshape(1, HID)

    degp = _sc_degree(edge_index)
    g1, dinv = _mm_scale(x, W1, degp)
    acc1 = _sc_edge(g1, edge_index, HID)
    g2 = _mid(acc1, g1, dinv, b1r, gammar, betar, w2p)
    acc2 = _sc_edge(g2, edge_index, F2)
    return _final(acc2, g2, dinv, b2r)
